# baseline (speedup 1.0000x reference)
import functools

import jax
import jax.numpy as jnp
from jax import lax
from jax.experimental import pallas as pl
from jax.experimental.pallas import tpu as pltpu
from jax.experimental.pallas import tpu_sc as plsc

_NUM_WORKERS = 32
_CHUNK = 128


@jax.jit
def _gather_sc(tensor, index):
    b = index.shape[0]
    _, d = tensor.shape
    b_per_w = b // _NUM_WORKERS
    n_chunks = b_per_w // _CHUNK
    mesh = plsc.VectorSubcoreMesh(core_axis_name="c", subcore_axis_name="s")

    @functools.partial(
        pl.kernel,
        mesh=mesh,
        out_type=jax.ShapeDtypeStruct((b, d), jnp.float32),
        scratch_types=[
            pltpu.VMEM((b_per_w,), jnp.int32),
            pltpu.VMEM((b_per_w, d), jnp.float32),
            pltpu.SemaphoreType.DMA,
        ],
        compiler_params=pltpu.CompilerParams(use_tc_tiling_on_sc=False),
    )
    def k(table_hbm, idx_hbm, out_hbm, idx_v, rows_v, sem):
        wid = lax.axis_index("s") * 2 + lax.axis_index("c")
        base = wid * b_per_w
        pltpu.sync_copy(idx_hbm.at[pl.ds(base, b_per_w)], idx_v)
        copies = []
        for j in range(n_chunks):
            copies.append(
                pltpu.async_copy(
                    table_hbm.at[idx_v.at[pl.ds(j * _CHUNK, _CHUNK)]],
                    rows_v.at[pl.ds(j * _CHUNK, _CHUNK)],
                    sem,
                )
            )
        for c in copies:
            c.wait()
        pltpu.sync_copy(rows_v, out_hbm.at[pl.ds(base, b_per_w)])

    return k(tensor, index)


def kernel(tensor, index):
    return _gather_sc(tensor, index.reshape(-1).astype(jnp.int32))


# speedup vs baseline: 1.7284x; 1.7284x over previous
import functools

import jax
import jax.numpy as jnp
from jax import lax
from jax.experimental import pallas as pl
from jax.experimental.pallas import tpu as pltpu
from jax.experimental.pallas import tpu_sc as plsc

_NUM_WORKERS = 32
_L = 16


@jax.jit
def _gather_sc(tensor, idx):
    b = idx.shape[0]
    d = tensor.shape[1]
    b_per_w = b // _NUM_WORKERS
    n_groups = b_per_w // _L
    mesh = plsc.VectorSubcoreMesh(core_axis_name="c", subcore_axis_name="s")

    @functools.partial(
        pl.kernel,
        mesh=mesh,
        out_type=jax.ShapeDtypeStruct((b, d), jnp.float32),
        scratch_types=[
            pltpu.VMEM((b_per_w,), jnp.int32),
            pltpu.VMEM((b_per_w, d), jnp.float32),
            pltpu.SemaphoreType.DMA,
        ],
    )
    def k(table_hbm, idx_hbm, out_hbm, idx_v, rows_v, sem):
        wid = lax.axis_index("s") * 2 + lax.axis_index("c")
        base = wid * b_per_w
        pltpu.sync_copy(idx_hbm.at[pl.ds(base, b_per_w)], idx_v)

        def fire_group(g, _):
            vec = idx_v[pl.ds(g * _L, _L)]
            for j in range(_L):
                r = vec[j]
                pltpu.async_copy(
                    table_hbm.at[pl.ds(r, 1), :],
                    rows_v.at[pl.ds(g * _L + j, 1), :],
                    sem,
                )
            return 0

        lax.fori_loop(0, n_groups, fire_group, 0)
        pltpu.make_async_copy(
            table_hbm.at[pl.ds(0, b_per_w), :], rows_v, sem
        ).wait()
        pltpu.sync_copy(rows_v, out_hbm.at[pl.ds(base, b_per_w)])

    return k(tensor, idx)


def kernel(tensor, index):
    idx = index.reshape(-1).astype(jnp.int32)
    return _gather_sc(tensor, idx)


# speedup vs baseline: 1.7370x; 1.0050x over previous
import functools

import jax
import jax.numpy as jnp
from jax import lax
from jax.experimental import pallas as pl
from jax.experimental.pallas import tpu as pltpu
from jax.experimental.pallas import tpu_sc as plsc

_NUM_WORKERS = 32
_L = 16


@jax.jit
def _gather_sc(tensor, idx):
    b = idx.shape[0]
    d = tensor.shape[1]
    b_per_w = b // _NUM_WORKERS
    n_groups = b_per_w // _L
    mesh = plsc.VectorSubcoreMesh(core_axis_name="c", subcore_axis_name="s")

    @functools.partial(
        pl.kernel,
        mesh=mesh,
        out_type=jax.ShapeDtypeStruct((b, d), jnp.float32),
        scratch_types=[
            pltpu.VMEM((b_per_w,), jnp.int32),
            pltpu.VMEM((b_per_w, d), jnp.float32),
            pltpu.SemaphoreType.DMA,
        ],
    )
    def k(table_hbm, idx_hbm, out_hbm, idx_v, rows_v, sem):
        wid = lax.axis_index("s") * 2 + lax.axis_index("c")
        base = wid * b_per_w
        pltpu.sync_copy(idx_hbm.at[pl.ds(base, b_per_w)], idx_v)

        @plsc.parallel_loop(0, n_groups, unroll=2)
        def fire_group(g):
            vec = idx_v[pl.ds(g * _L, _L)]
            for j in range(_L):
                r = vec[j]
                pltpu.async_copy(
                    table_hbm.at[pl.ds(r, 1), :],
                    rows_v.at[pl.ds(g * _L + j, 1), :],
                    sem,
                )
        pltpu.make_async_copy(
            table_hbm.at[pl.ds(0, b_per_w), :], rows_v, sem
        ).wait()
        pltpu.sync_copy(rows_v, out_hbm.at[pl.ds(base, b_per_w)])

    return k(tensor, idx)


def kernel(tensor, index):
    idx = index.reshape(-1).astype(jnp.int32)
    return _gather_sc(tensor, idx)


# speedup vs baseline: 2.2247x; 1.2808x over previous
import functools

import jax
import jax.numpy as jnp
from jax import lax
from jax.experimental import pallas as pl
from jax.experimental.pallas import tpu as pltpu
from jax.experimental.pallas import tpu_sc as plsc

_NUM_WORKERS = 32
_L = 16


@jax.jit
def _gather_sc(table_t, sorted_r, order):
    d, _ = table_t.shape
    b = sorted_r.shape[0]
    b_per_w = b // _NUM_WORKERS
    n_groups = b_per_w // _L
    mesh = plsc.VectorSubcoreMesh(core_axis_name="c", subcore_axis_name="s")

    @functools.partial(
        pl.kernel,
        mesh=mesh,
        out_type=jax.ShapeDtypeStruct((b, d), jnp.float32),
        scratch_types=[
            pltpu.VMEM((b_per_w,), jnp.int32),
            pltpu.VMEM((b_per_w,), jnp.int32),
            pltpu.VMEM((d, 128), jnp.float32),
            pltpu.VMEM((_L, d), jnp.float32),
            pltpu.SemaphoreType.DMA,
        ],
        compiler_params=pltpu.CompilerParams(needs_layout_passes=False),
    )
    def k(table_hbm, srt_hbm, ord_hbm, out_hbm, srt_v, ord_v, blk_v, row_v, sem):
        wid = lax.axis_index("s") * 2 + lax.axis_index("c")
        base = wid * b_per_w
        pltpu.sync_copy(srt_hbm.at[pl.ds(base, b_per_w)], srt_v)
        pltpu.sync_copy(ord_hbm.at[pl.ds(base, b_per_w)], ord_v)
        lanes = [jnp.arange(_L, dtype=jnp.int32) + _L * kk for kk in range(d // _L)]

        def group(g, cur_blk):
            rvec = srt_v[pl.ds(g * _L, _L)]
            pvec = ord_v[pl.ds(g * _L, _L)]
            for j in range(_L):
                r = rvec[j]
                p = pvec[j]
                blk = r >> 7
                c = r & 127

                @pl.when(blk != cur_blk)
                def _():
                    pltpu.sync_copy(
                        table_hbm.at[:, pl.ds(pl.multiple_of(blk * 128, 128), 128)],
                        blk_v,
                    )

                cur_blk = blk
                cvec = jnp.full((_L,), c, dtype=jnp.int32)
                for kk in range(d // _L):
                    row_v[j, pl.ds(kk * _L, _L)] = plsc.load_gather(
                        blk_v, [lanes[kk], cvec]
                    )
                pltpu.async_copy(
                    row_v.at[pl.ds(j, 1), :], out_hbm.at[pl.ds(p, 1), :], sem
                )
            pltpu.make_async_copy(out_hbm.at[pl.ds(0, _L), :], row_v, sem).wait()
            return cur_blk

        lax.fori_loop(0, n_groups, group, jnp.int32(-1))

    return k(table_t, sorted_r, order)


def kernel(tensor, index):
    idx = index.reshape(-1).astype(jnp.int32)
    pos = jnp.arange(idx.shape[0], dtype=jnp.int32)
    sorted_r, order = lax.sort((idx, pos), num_keys=1)
    return _gather_sc(tensor.T, sorted_r, order)


# speedup vs baseline: 4.3152x; 1.9396x over previous
import functools

import jax
import jax.numpy as jnp
from jax import lax
from jax.experimental import pallas as pl
from jax.experimental.pallas import tpu as pltpu
from jax.experimental.pallas import tpu_sc as plsc

_NUM_WORKERS = 32
_L = 16
_NB = 4


@jax.jit
def _gather_sc(table_t, sorted_r, order, lord, dist):
    d, _ = table_t.shape
    b = sorted_r.shape[0]
    b_per_w = b // _NUM_WORKERS
    n_groups = b_per_w // _L
    nk = d // _L
    mesh = plsc.VectorSubcoreMesh(core_axis_name="c", subcore_axis_name="s")

    @functools.partial(
        pl.kernel,
        mesh=mesh,
        out_type=jax.ShapeDtypeStruct((b, d), jnp.float32),
        scratch_types=[
            pltpu.VMEM((b_per_w,), jnp.int32),
            pltpu.VMEM((b_per_w,), jnp.int32),
            pltpu.VMEM((b_per_w,), jnp.int32),
            pltpu.VMEM((b_per_w,), jnp.int32),
            pltpu.VMEM((_NB, d, 128), jnp.float32),
            pltpu.VMEM((_L, d), jnp.float32),
            [pltpu.SemaphoreType.DMA] * _NB,
            pltpu.SemaphoreType.DMA,
        ],
        compiler_params=pltpu.CompilerParams(needs_layout_passes=False),
    )
    def k(table_hbm, srt_hbm, ord_hbm, lord_hbm, dist_hbm, out_hbm,
          srt_v, ord_v, lord_v, dist_v, blk_v, row_v, bsems, osem):
        wid = lax.axis_index("s") * 2 + lax.axis_index("c")
        base = wid * b_per_w
        pltpu.sync_copy(srt_hbm.at[pl.ds(base, b_per_w)], srt_v)
        pltpu.sync_copy(ord_hbm.at[pl.ds(base, b_per_w)], ord_v)
        pltpu.sync_copy(lord_hbm.at[pl.ds(base, b_per_w)], lord_v)
        pltpu.sync_copy(dist_hbm.at[pl.ds(base, b_per_w)], dist_v)
        lanes = [jnp.arange(_L, dtype=jnp.int32) + _L * kk for kk in range(nk)]
        iota = jnp.arange(_L, dtype=jnp.int32)

        def fetch(blkid, q):
            pltpu.async_copy(
                table_hbm.at[:, pl.ds(pl.multiple_of(blkid * 128, 128), 128)],
                blk_v.at[q],
                bsems[q],
            )

        def dist_at(n):
            nb16 = pl.multiple_of((n >> 4) << 4, _L)
            dvec = dist_v[pl.ds(nb16, _L)]
            return jnp.sum(jnp.where(iota == (n & (_L - 1)), dvec, 0))

        dvec0 = dist_v[pl.ds(0, _L)]
        for q in range(_NB - 1):
            fetch(dvec0[q], q)

        def group(g, cur):
            rvec = srt_v[pl.ds(g * _L, _L)]
            pvec = ord_v[pl.ds(g * _L, _L)]
            lvec = lord_v[pl.ds(g * _L, _L)]
            for j in range(_L):
                r = rvec[j]
                p = pvec[j]
                lo = lvec[j]
                c = r & 127

                @pl.when(lo != cur)
                def _():
                    nxt = jnp.minimum(lo + _NB - 1, b_per_w - 1)
                    blk_nxt = dist_at(nxt)
                    for qq in range(_NB):
                        @pl.when((lo & (_NB - 1)) == qq)
                        def _():
                            pltpu.make_async_copy(
                                table_hbm.at[:, pl.ds(0, 128)],
                                blk_v.at[qq],
                                bsems[qq],
                            ).wait()
                            fetch(blk_nxt, (qq + _NB - 1) % _NB)

                cur = jnp.where(lo != cur, lo, cur)
                cvec = jnp.full((_L,), c, dtype=jnp.int32)
                qvec = jnp.full((_L,), lo & (_NB - 1), dtype=jnp.int32)
                for kk in range(nk):
                    row_v[j, pl.ds(kk * _L, _L)] = plsc.load_gather(
                        blk_v, [qvec, lanes[kk], cvec]
                    )
                pltpu.async_copy(
                    row_v.at[pl.ds(j, 1), :], out_hbm.at[pl.ds(p, 1), :], osem
                )
            pltpu.make_async_copy(out_hbm.at[pl.ds(0, _L), :], row_v, osem).wait()
            return cur

        cur = lax.fori_loop(0, n_groups, group, jnp.int32(-1))
        for qq in range(_NB):
            @pl.when((cur & (_NB - 1)) != qq)
            def _():
                pltpu.make_async_copy(
                    table_hbm.at[:, pl.ds(0, 128)], blk_v.at[qq], bsems[qq]
                ).wait()

    return k(table_t, sorted_r, order, lord, dist)


def kernel(tensor, index):
    idx = index.reshape(-1).astype(jnp.int32)
    n = idx.shape[0]
    bpw = n // _NUM_WORKERS
    pos = jnp.arange(n, dtype=jnp.int32)
    sorted_r, order = lax.sort((idx, pos), num_keys=1)
    blk = sorted_r >> 7
    seg_first = (pos % bpw) == 0
    newb = jnp.concatenate([jnp.ones((1,), bool), blk[1:] != blk[:-1]]) | seg_first
    nb32 = newb.astype(jnp.int32).reshape(_NUM_WORKERS, bpw)
    lord = (jnp.cumsum(nb32, axis=1) - 1).reshape(-1).astype(jnp.int32)
    big = jnp.int32(1 << 20)
    dist = jnp.sort(
        jnp.where(newb, blk, big).reshape(_NUM_WORKERS, bpw), axis=1
    ).reshape(-1)
    nblk = (tensor.shape[0] + 127) // 128
    dist = jnp.minimum(dist, nblk - 1).astype(jnp.int32)
    return _gather_sc(tensor.T, sorted_r, order, lord, dist)
